# baseline (speedup 1.0000x reference)
import functools

import jax
import jax.numpy as jnp
from jax import lax
from jax.experimental import pallas as pl
from jax.experimental.pallas import tpu as pltpu
from jax.experimental.pallas import tpu_sc as plsc

_NC = 2
_NS = 16
_L = 16
_NW = _NC * _NS
_H = 512
_W = 512
_HW = _H * _W
_CH = 32768
_NCH = _HW // _CH
_RPC = _CH // _W
_VIT = _CH // _L
_UNR = 8
_LN2 = 0.6931471805599453

_QSC = 32
_QTC = 32
_TCB = 4
_TBR = _H // _TCB


def _sc_body(scores_hbm, off_hbm, scl_hbm, out_hbm,
             buf0, buf1, row0, row1, row2, resbuf, sem0, sem1, semr):
    wid = lax.axis_index("s") * _NC + lax.axis_index("c")
    lane = lax.iota(jnp.int32, _L)
    bufs = (buf0, buf1)
    sems = (sem0, sem1)

    def start(c):
        src = scores_hbm.at[pl.ds(wid * _H + c * _RPC, _RPC), :]
        return pltpu.async_copy(src, bufs[c % 2], sems[c % 2])

    cps = [None] * _NCH
    cps[0] = start(0)
    cps[1] = start(1)

    neg_inf = jnp.full((_L,), -jnp.inf, jnp.float32)
    zero_i = jnp.zeros((_L,), jnp.int32)
    ms = [neg_inf] * _UNR
    ixs = [zero_i] * _UNR

    for c in range(_NCH):
        cps[c].wait()
        buf = bufs[c % 2]
        cbase = c * _VIT

        def inner(i, carry, buf=buf, cbase=cbase):
            m, ix = list(carry[0]), list(carry[1])
            r = i >> 2
            cw = (i & 3) * (_UNR * _L)
            for k in range(_UNR):
                v = buf[r, pl.ds(cw + k * _L, _L)]
                g = v > m[k]
                m[k] = jnp.where(g, v, m[k])
                ix[k] = jnp.where(g, cbase + i * _UNR + k, ix[k])
            return tuple(m), tuple(ix)

        acc = lax.fori_loop(0, _VIT // _UNR, inner, (tuple(ms), tuple(ixs)))
        ms, ixs = list(acc[0]), list(acc[1])
        if c + 2 < _NCH:
            cps[c + 2] = start(c + 2)

    pairs = [(ms[k], ixs[k] * _L + lane) for k in range(_UNR)]
    while len(pairs) > 1:
        nxt = []
        for a in range(0, len(pairs), 2):
            (m1, f1), (m2, f2) = pairs[a], pairs[a + 1]
            take = (m2 > m1) | ((m2 == m1) & (f2 < f1))
            nxt.append((jnp.where(take, m2, m1), jnp.where(take, f2, f1)))
        pairs = nxt
    cmax, flat = pairs[0]
    m = jnp.max(cmax)
    cand = jnp.where(cmax == m, flat, jnp.int32(2**31 - 1))
    idx = jnp.min(cand)
    h = idx >> 9
    w = idx & (_W - 1)
    cp0 = pltpu.async_copy(off_hbm.at[(wid * 2) * _H + h], row0, semr)
    cp1 = pltpu.async_copy(off_hbm.at[(wid * 2 + 1) * _H + h], row1, semr)
    cp2 = pltpu.async_copy(scl_hbm.at[wid * _H + h], row2, semr)
    cp0.wait()
    cp1.wait()
    cp2.wait()
    wv = jnp.full((_L,), w, jnp.int32)
    o0 = plsc.load_gather(row0, [wv])
    o1 = plsc.load_gather(row1, [wv])
    sv = plsc.load_gather(row2, [wv])
    xs = w.astype(jnp.float32) + o0
    ys = h.astype(jnp.float32) + o1
    sc = jnp.exp(sv * jnp.float32(_LN2))
    res = jnp.zeros((_L,), jnp.float32)
    res = jnp.where(lane == 0, xs, res)
    res = jnp.where(lane == 1, ys, res)
    res = jnp.where(lane == 2, sc, res)
    resbuf[...] = res
    pltpu.sync_copy(resbuf, out_hbm.at[pl.ds(wid * _L, _L)])


_TUNR = 4


def _tc_argmax_body(s0, s1, s2, s3, idx_ref):
    rows8 = lax.broadcasted_iota(jnp.int32, (8, 128), 0)
    cols128 = lax.broadcasted_iota(jnp.int32, (8, 128), 1)
    neg = jnp.full((8, 128), -jnp.inf, jnp.float32)
    zero = jnp.zeros((8, 128), jnp.int32)

    acc = ((neg,) * _TUNR, (zero,) * _TUNR)
    for j, sref in enumerate((s0, s1, s2, s3)):

        def inner(a, carry, sref=sref, j=j):
            m, ix = list(carry[0]), list(carry[1])
            for c in range(_TUNR):
                v = sref[pl.ds(a * 8, 8), pl.ds(c * 128, 128)]
                g = v > m[c]
                m[c] = jnp.where(g, v, m[c])
                ix[c] = jnp.where(g, j * (_H // 32) + a, ix[c])
            return tuple(m), tuple(ix)

        acc = lax.fori_loop(0, _H // 32, inner, acc)
    pairs = []
    for c in range(_TUNR):
        flat = (acc[0][c],
                (acc[1][c] * 8 + rows8) * _W + c * 128 + cols128)
        pairs.append(flat)
    while len(pairs) > 1:
        nxt = []
        for a in range(0, len(pairs), 2):
            (m1, f1), (m2, f2) = pairs[a], pairs[a + 1]
            take = (m2 > m1) | ((m2 == m1) & (f2 < f1))
            nxt.append((jnp.where(take, m2, m1), jnp.where(take, f2, f1)))
        pairs = nxt
    m8, f8 = pairs[0]
    mx = jnp.max(m8)
    cand = jnp.where(m8 == mx, f8, jnp.int32(2**31 - 1))
    idx = jnp.min(cand)
    i128 = lax.broadcasted_iota(jnp.int32, (128,), 0)
    idx_ref[0, 0, :] = jnp.where(i128 == 0, idx, 0)


def _tc_gather_body(idx_smem, idx_vmem, off_hbm, scl_hbm, out_ref,
                    rows0, rows1, rows2, sem):
    copies = []
    for q in range(_QTC):
        idx = idx_smem[q]
        h = idx >> 9
        qg = _QSC + q
        copies.append(pltpu.make_async_copy(
            off_hbm.at[pl.ds((qg * 2) * _H + h, 1), :],
            rows0.at[pl.ds(q, 1), :], sem))
        copies.append(pltpu.make_async_copy(
            off_hbm.at[pl.ds((qg * 2 + 1) * _H + h, 1), :],
            rows1.at[pl.ds(q, 1), :], sem))
        copies.append(pltpu.make_async_copy(
            scl_hbm.at[pl.ds(qg * _H + h, 1), :],
            rows2.at[pl.ds(q, 1), :], sem))
    for cp in copies:
        cp.start()
    for cp in copies:
        cp.wait()
    iw = lax.broadcasted_iota(jnp.int32, (_QTC, _W), 1)
    idxv = idx_vmem[:, 0, 0:1]
    wq = idxv & (_W - 1)
    hq = idxv >> 9
    sel = (iw == wq).astype(jnp.float32)
    o0 = jnp.sum(rows0[...] * sel, axis=1, keepdims=True)
    o1 = jnp.sum(rows1[...] * sel, axis=1, keepdims=True)
    sv = jnp.sum(rows2[...] * sel, axis=1, keepdims=True)
    xs = wq.astype(jnp.float32) + o0
    ys = hq.astype(jnp.float32) + o1
    sc = jnp.exp(sv * jnp.float32(_LN2))
    i128 = lax.broadcasted_iota(jnp.int32, (_QTC, 128), 1)
    vals = jnp.where(i128 == 0, xs,
                     jnp.where(i128 == 1, ys,
                               jnp.where(i128 == 2, sc, 0.0)))
    out_ref[...] = vals


@jax.jit
def _detect(scores2, off2, scl2):
    mesh = plsc.VectorSubcoreMesh(
        core_axis_name="c", subcore_axis_name="s",
        num_cores=_NC, num_subcores=_NS)
    sc_run = functools.partial(
        pl.kernel,
        out_type=jax.ShapeDtypeStruct((_NW * _L,), jnp.float32),
        mesh=mesh,
        scratch_types=[
            pltpu.VMEM((_RPC, _W), jnp.float32),
            pltpu.VMEM((_RPC, _W), jnp.float32),
            pltpu.VMEM((_W,), jnp.float32),
            pltpu.VMEM((_W,), jnp.float32),
            pltpu.VMEM((_W,), jnp.float32),
            pltpu.VMEM((_L,), jnp.float32),
            pltpu.SemaphoreType.DMA,
            pltpu.SemaphoreType.DMA,
            pltpu.SemaphoreType.DMA,
        ],
        compiler_params=pltpu.CompilerParams(needs_layout_passes=False),
    )(_sc_body)
    out_sc = sc_run(scores2, off2, scl2)

    idx_tc = pl.pallas_call(
        _tc_argmax_body,
        grid=(_QTC,),
        in_specs=[
            pl.BlockSpec((_H // 4, _W),
                         lambda q, j=j: ((_QSC + q) * 4 + j, 0))
            for j in range(4)
        ],
        out_specs=pl.BlockSpec((1, 1, 128), lambda q: (q, 0, 0)),
        out_shape=jax.ShapeDtypeStruct((_QTC, 1, 128), jnp.int32),
    )(scores2, scores2, scores2, scores2)

    out_tc = pl.pallas_call(
        _tc_gather_body,
        in_specs=[
            pl.BlockSpec(memory_space=pltpu.SMEM),
            pl.BlockSpec((_QTC, 1, 128), lambda: (0, 0, 0)),
            pl.BlockSpec(memory_space=pl.ANY),
            pl.BlockSpec(memory_space=pl.ANY),
        ],
        out_specs=pl.BlockSpec((_QTC, 128), lambda: (0, 0)),
        out_shape=jax.ShapeDtypeStruct((_QTC, 128), jnp.float32),
        scratch_shapes=[
            pltpu.VMEM((_QTC, _W), jnp.float32),
            pltpu.VMEM((_QTC, _W), jnp.float32),
            pltpu.VMEM((_QTC, _W), jnp.float32),
            pltpu.SemaphoreType.DMA,
        ],
    )(idx_tc[:, 0, 0], idx_tc, off2, scl2)

    r_sc = out_sc.reshape(_QSC, _L)[:, :8]
    r_tc = out_tc[:, :8]
    return jnp.concatenate([r_sc, r_tc], axis=0)


def kernel(scores, scales, offsets, pool_ratio):
    qn = scores.shape[0]
    scores2 = scores.reshape(qn * _H, _W)
    off2 = offsets.reshape(qn * 2 * _H, _W)
    scl2 = scales.reshape(qn * _H, _W)
    r = _detect(scores2, off2, scl2)
    pf = jnp.asarray(pool_ratio, jnp.float32)
    positions = (r[:, :2] + 0.5) * pf - 0.5
    sel_scales = r[:, 2]
    return positions, sel_scales


# speedup vs baseline: 1.1645x; 1.1645x over previous
import functools

import jax
import jax.numpy as jnp
from jax import lax
from jax.experimental import pallas as pl
from jax.experimental.pallas import tpu as pltpu
from jax.experimental.pallas import tpu_sc as plsc

_NC = 2
_NS = 16
_L = 16
_NW = _NC * _NS
_H = 512
_W = 512
_HW = _H * _W
_CH = 32768
_NCH = _HW // _CH
_RPC = _CH // _W
_VIT = _CH // _L
_UNR = 8
_LN2 = 0.6931471805599453

_QSC = 32
_QTC = 32
_TCB = 4
_TBR = _H // _TCB


def _sc_body(scores_hbm, off_hbm, scl_hbm, out_hbm,
             buf0, buf1, row0, row1, row2, resbuf, sem0, sem1, semr):
    wid = lax.axis_index("s") * _NC + lax.axis_index("c")
    lane = lax.iota(jnp.int32, _L)
    bufs = (buf0, buf1)
    sems = (sem0, sem1)

    def start(c):
        src = scores_hbm.at[pl.ds(wid * _H + c * _RPC, _RPC), :]
        return pltpu.async_copy(src, bufs[c % 2], sems[c % 2])

    cps = [None] * _NCH
    cps[0] = start(0)
    cps[1] = start(1)

    neg_inf = jnp.full((_L,), -jnp.inf, jnp.float32)
    zero_i = jnp.zeros((_L,), jnp.int32)
    ms = [neg_inf] * _UNR
    ixs = [zero_i] * _UNR

    for c in range(_NCH):
        cps[c].wait()
        buf = bufs[c % 2]
        cbase = c * _VIT

        def inner(i, carry, buf=buf, cbase=cbase):
            m, ix = list(carry[0]), list(carry[1])
            r = i >> 2
            cw = (i & 3) * (_UNR * _L)
            for k in range(_UNR):
                v = buf[r, pl.ds(cw + k * _L, _L)]
                g = v > m[k]
                m[k] = jnp.where(g, v, m[k])
                ix[k] = jnp.where(g, cbase + i * _UNR + k, ix[k])
            return tuple(m), tuple(ix)

        acc = lax.fori_loop(0, _VIT // _UNR, inner, (tuple(ms), tuple(ixs)))
        ms, ixs = list(acc[0]), list(acc[1])
        if c + 2 < _NCH:
            cps[c + 2] = start(c + 2)

    pairs = [(ms[k], ixs[k] * _L + lane) for k in range(_UNR)]
    while len(pairs) > 1:
        nxt = []
        for a in range(0, len(pairs), 2):
            (m1, f1), (m2, f2) = pairs[a], pairs[a + 1]
            take = (m2 > m1) | ((m2 == m1) & (f2 < f1))
            nxt.append((jnp.where(take, m2, m1), jnp.where(take, f2, f1)))
        pairs = nxt
    cmax, flat = pairs[0]
    m = jnp.max(cmax)
    cand = jnp.where(cmax == m, flat, jnp.int32(2**31 - 1))
    idx = jnp.min(cand)
    h = idx >> 9
    w = idx & (_W - 1)
    cp0 = pltpu.async_copy(off_hbm.at[(wid * 2) * _H + h], row0, semr)
    cp1 = pltpu.async_copy(off_hbm.at[(wid * 2 + 1) * _H + h], row1, semr)
    cp2 = pltpu.async_copy(scl_hbm.at[wid * _H + h], row2, semr)
    cp0.wait()
    cp1.wait()
    cp2.wait()
    wv = jnp.full((_L,), w, jnp.int32)
    o0 = plsc.load_gather(row0, [wv])
    o1 = plsc.load_gather(row1, [wv])
    sv = plsc.load_gather(row2, [wv])
    xs = w.astype(jnp.float32) + o0
    ys = h.astype(jnp.float32) + o1
    sc = jnp.exp(sv * jnp.float32(_LN2))
    res = jnp.zeros((_L,), jnp.float32)
    res = jnp.where(lane == 0, xs, res)
    res = jnp.where(lane == 1, ys, res)
    res = jnp.where(lane == 2, sc, res)
    resbuf[...] = res
    pltpu.sync_copy(resbuf, out_hbm.at[pl.ds(wid * _L, _L)])


_TUNR = 4


_TRING = 8
_TCCH = 128


def _tc_argmax_body(scores_hbm, idx_ref, *rest):
    bufs = rest[:_TRING]
    sems = rest[_TRING:]
    rows8 = lax.broadcasted_iota(jnp.int32, (8, 128), 0)
    cols128 = lax.broadcasted_iota(jnp.int32, (8, 128), 1)
    neg = jnp.full((8, 128), -jnp.inf, jnp.float32)
    zero = jnp.zeros((8, 128), jnp.int32)
    i128 = lax.broadcasted_iota(jnp.int32, (128,), 0)

    ntask = _QTC * 4

    def start(t):
        row0 = (_QSC + t // 4) * _H + (t % 4) * _TCCH
        return pltpu.make_async_copy(
            scores_hbm.at[pl.ds(row0, _TCCH), :], bufs[t % _TRING],
            sems[t % _TRING])

    cps = [None] * ntask
    for t in range(_TRING):
        cps[t] = start(t)
        cps[t].start()

    acc = ((neg,) * _TUNR, (zero,) * _TUNR)
    for t in range(ntask):
        j = t % 4
        cps[t].wait()
        buf = bufs[t % _TRING]

        def inner(a, carry, buf=buf, j=j):
            m, ix = list(carry[0]), list(carry[1])
            for c in range(_TUNR):
                v = buf[pl.ds(a * 8, 8), pl.ds(c * 128, 128)]
                g = v > m[c]
                m[c] = jnp.where(g, v, m[c])
                ix[c] = jnp.where(g, j * (_TCCH // 8) + a, ix[c])
            return tuple(m), tuple(ix)

        acc = lax.fori_loop(0, _TCCH // 8, inner, acc)
        if t + _TRING < ntask:
            cps[t + _TRING] = start(t + _TRING)
            cps[t + _TRING].start()
        if j == 3:
            pairs = []
            for c in range(_TUNR):
                pairs.append((acc[0][c],
                              (acc[1][c] * 8 + rows8) * _W
                              + c * 128 + cols128))
            while len(pairs) > 1:
                nxt = []
                for a in range(0, len(pairs), 2):
                    (m1, f1), (m2, f2) = pairs[a], pairs[a + 1]
                    take = (m2 > m1) | ((m2 == m1) & (f2 < f1))
                    nxt.append((jnp.where(take, m2, m1),
                                jnp.where(take, f2, f1)))
                pairs = nxt
            m8, f8 = pairs[0]
            mx = jnp.max(m8)
            cand = jnp.where(m8 == mx, f8, jnp.int32(2**31 - 1))
            idx = jnp.min(cand)
            idx_ref[t // 4, 0, :] = jnp.where(i128 == 0, idx, 0)
            acc = ((neg,) * _TUNR, (zero,) * _TUNR)


def _tc_gather_body(idx_smem, idx_vmem, off_hbm, scl_hbm, out_ref,
                    rows0, rows1, rows2, sem):
    copies = []
    for q in range(_QTC):
        idx = idx_smem[q]
        h = idx >> 9
        qg = _QSC + q
        copies.append(pltpu.make_async_copy(
            off_hbm.at[pl.ds((qg * 2) * _H + h, 1), :],
            rows0.at[pl.ds(q, 1), :], sem))
        copies.append(pltpu.make_async_copy(
            off_hbm.at[pl.ds((qg * 2 + 1) * _H + h, 1), :],
            rows1.at[pl.ds(q, 1), :], sem))
        copies.append(pltpu.make_async_copy(
            scl_hbm.at[pl.ds(qg * _H + h, 1), :],
            rows2.at[pl.ds(q, 1), :], sem))
    for cp in copies:
        cp.start()
    for cp in copies:
        cp.wait()
    iw = lax.broadcasted_iota(jnp.int32, (_QTC, _W), 1)
    idxv = idx_vmem[:, 0, 0:1]
    wq = idxv & (_W - 1)
    hq = idxv >> 9
    sel = (iw == wq).astype(jnp.float32)
    o0 = jnp.sum(rows0[...] * sel, axis=1, keepdims=True)
    o1 = jnp.sum(rows1[...] * sel, axis=1, keepdims=True)
    sv = jnp.sum(rows2[...] * sel, axis=1, keepdims=True)
    xs = wq.astype(jnp.float32) + o0
    ys = hq.astype(jnp.float32) + o1
    sc = jnp.exp(sv * jnp.float32(_LN2))
    i128 = lax.broadcasted_iota(jnp.int32, (_QTC, 128), 1)
    vals = jnp.where(i128 == 0, xs,
                     jnp.where(i128 == 1, ys,
                               jnp.where(i128 == 2, sc, 0.0)))
    out_ref[...] = vals


@jax.jit
def _detect(scores2, off2, scl2):
    mesh = plsc.VectorSubcoreMesh(
        core_axis_name="c", subcore_axis_name="s",
        num_cores=_NC, num_subcores=_NS)
    sc_run = functools.partial(
        pl.kernel,
        out_type=jax.ShapeDtypeStruct((_NW * _L,), jnp.float32),
        mesh=mesh,
        scratch_types=[
            pltpu.VMEM((_RPC, _W), jnp.float32),
            pltpu.VMEM((_RPC, _W), jnp.float32),
            pltpu.VMEM((_W,), jnp.float32),
            pltpu.VMEM((_W,), jnp.float32),
            pltpu.VMEM((_W,), jnp.float32),
            pltpu.VMEM((_L,), jnp.float32),
            pltpu.SemaphoreType.DMA,
            pltpu.SemaphoreType.DMA,
            pltpu.SemaphoreType.DMA,
        ],
        compiler_params=pltpu.CompilerParams(needs_layout_passes=False),
    )(_sc_body)
    out_sc = sc_run(scores2, off2, scl2)

    idx_tc = pl.pallas_call(
        _tc_argmax_body,
        in_specs=[pl.BlockSpec(memory_space=pl.ANY)],
        out_specs=pl.BlockSpec((_QTC, 1, 128), lambda: (0, 0, 0)),
        out_shape=jax.ShapeDtypeStruct((_QTC, 1, 128), jnp.int32),
        scratch_shapes=(
            [pltpu.VMEM((_TCCH, _W), jnp.float32)] * _TRING
            + [pltpu.SemaphoreType.DMA] * _TRING
        ),
    )(scores2)

    out_tc = pl.pallas_call(
        _tc_gather_body,
        in_specs=[
            pl.BlockSpec(memory_space=pltpu.SMEM),
            pl.BlockSpec((_QTC, 1, 128), lambda: (0, 0, 0)),
            pl.BlockSpec(memory_space=pl.ANY),
            pl.BlockSpec(memory_space=pl.ANY),
        ],
        out_specs=pl.BlockSpec((_QTC, 128), lambda: (0, 0)),
        out_shape=jax.ShapeDtypeStruct((_QTC, 128), jnp.float32),
        scratch_shapes=[
            pltpu.VMEM((_QTC, _W), jnp.float32),
            pltpu.VMEM((_QTC, _W), jnp.float32),
            pltpu.VMEM((_QTC, _W), jnp.float32),
            pltpu.SemaphoreType.DMA,
        ],
    )(idx_tc[:, 0, 0], idx_tc, off2, scl2)

    r_sc = out_sc.reshape(_QSC, _L)[:, :8]
    r_tc = out_tc[:, :8]
    return jnp.concatenate([r_sc, r_tc], axis=0)


def kernel(scores, scales, offsets, pool_ratio):
    qn = scores.shape[0]
    scores2 = scores.reshape(qn * _H, _W)
    off2 = offsets.reshape(qn * 2 * _H, _W)
    scl2 = scales.reshape(qn * _H, _W)
    r = _detect(scores2, off2, scl2)
    pf = jnp.asarray(pool_ratio, jnp.float32)
    positions = (r[:, :2] + 0.5) * pf - 0.5
    sel_scales = r[:, 2]
    return positions, sel_scales


# speedup vs baseline: 1.1879x; 1.0201x over previous
import functools

import jax
import jax.numpy as jnp
from jax import lax
from jax.experimental import pallas as pl
from jax.experimental.pallas import tpu as pltpu
from jax.experimental.pallas import tpu_sc as plsc

_NC = 2
_NS = 16
_L = 16
_NW = _NC * _NS
_H = 512
_W = 512
_HW = _H * _W
_CH = 32768
_NCH = _HW // _CH
_RPC = _CH // _W
_VIT = _CH // _L
_UNR = 8
_LN2 = 0.6931471805599453

_QSC = 32
_QTC = 32
_TCB = 4
_TBR = _H // _TCB


def _sc_body(scores_hbm, off_hbm, scl_hbm, out_hbm,
             buf0, buf1, row0, row1, row2, resbuf, sem0, sem1, semr):
    wid = lax.axis_index("s") * _NC + lax.axis_index("c")
    lane = lax.iota(jnp.int32, _L)
    bufs = (buf0, buf1)
    sems = (sem0, sem1)

    def start(c):
        src = scores_hbm.at[pl.ds(wid * _H + c * _RPC, _RPC), :]
        return pltpu.async_copy(src, bufs[c % 2], sems[c % 2])

    cps = [None] * _NCH
    cps[0] = start(0)
    cps[1] = start(1)

    neg_inf = jnp.full((_L,), -jnp.inf, jnp.float32)
    zero_i = jnp.zeros((_L,), jnp.int32)
    ms = [neg_inf] * _UNR
    ixs = [zero_i] * _UNR

    for c in range(_NCH):
        cps[c].wait()
        buf = bufs[c % 2]
        cbase = c * _VIT

        def inner(i, carry, buf=buf, cbase=cbase):
            m, ix = list(carry[0]), list(carry[1])
            r = i >> 2
            cw = (i & 3) * (_UNR * _L)
            for k in range(_UNR):
                v = buf[r, pl.ds(cw + k * _L, _L)]
                g = v > m[k]
                m[k] = jnp.where(g, v, m[k])
                ix[k] = jnp.where(g, cbase + i * _UNR + k, ix[k])
            return tuple(m), tuple(ix)

        acc = lax.fori_loop(0, _VIT // _UNR, inner, (tuple(ms), tuple(ixs)))
        ms, ixs = list(acc[0]), list(acc[1])
        if c + 2 < _NCH:
            cps[c + 2] = start(c + 2)

    pairs = [(ms[k], ixs[k] * _L + lane) for k in range(_UNR)]
    while len(pairs) > 1:
        nxt = []
        for a in range(0, len(pairs), 2):
            (m1, f1), (m2, f2) = pairs[a], pairs[a + 1]
            take = (m2 > m1) | ((m2 == m1) & (f2 < f1))
            nxt.append((jnp.where(take, m2, m1), jnp.where(take, f2, f1)))
        pairs = nxt
    cmax, flat = pairs[0]
    m = jnp.max(cmax)
    cand = jnp.where(cmax == m, flat, jnp.int32(2**31 - 1))
    idx = jnp.min(cand)
    h = idx >> 9
    w = idx & (_W - 1)
    cp0 = pltpu.async_copy(off_hbm.at[(wid * 2) * _H + h], row0, semr)
    cp1 = pltpu.async_copy(off_hbm.at[(wid * 2 + 1) * _H + h], row1, semr)
    cp2 = pltpu.async_copy(scl_hbm.at[wid * _H + h], row2, semr)
    cp0.wait()
    cp1.wait()
    cp2.wait()
    wv = jnp.full((_L,), w, jnp.int32)
    o0 = plsc.load_gather(row0, [wv])
    o1 = plsc.load_gather(row1, [wv])
    sv = plsc.load_gather(row2, [wv])
    xs = w.astype(jnp.float32) + o0
    ys = h.astype(jnp.float32) + o1
    sc = jnp.exp(sv * jnp.float32(_LN2))
    res = jnp.zeros((_L,), jnp.float32)
    res = jnp.where(lane == 0, xs, res)
    res = jnp.where(lane == 1, ys, res)
    res = jnp.where(lane == 2, sc, res)
    resbuf[...] = res
    pltpu.sync_copy(resbuf, out_hbm.at[pl.ds(wid * _L, _L)])


_TUNR = 4


_TRING = 8
_TCCH = 128


def _tc_argmax_body(scores_hbm, idx_ref, *rest):
    idx_scalars = []
    bufs = rest[:_TRING]
    sems = rest[_TRING:]
    rows8 = lax.broadcasted_iota(jnp.int32, (8, 128), 0)
    cols128 = lax.broadcasted_iota(jnp.int32, (8, 128), 1)
    neg = jnp.full((8, 128), -jnp.inf, jnp.float32)
    zero = jnp.zeros((8, 128), jnp.int32)
    i128 = lax.broadcasted_iota(jnp.int32, (128,), 0)

    ntask = _QTC * 4

    def start(t):
        row0 = (_QSC + t // 4) * _H + (t % 4) * _TCCH
        return pltpu.make_async_copy(
            scores_hbm.at[pl.ds(row0, _TCCH), :], bufs[t % _TRING],
            sems[t % _TRING])

    cps = [None] * ntask
    for t in range(_TRING):
        cps[t] = start(t)
        cps[t].start()

    acc = ((neg,) * _TUNR, (zero,) * _TUNR)
    for t in range(ntask):
        j = t % 4
        cps[t].wait()
        buf = bufs[t % _TRING]

        def inner(a, carry, buf=buf, j=j):
            m, ix = list(carry[0]), list(carry[1])
            for c in range(_TUNR):
                v = buf[pl.ds(a * 8, 8), pl.ds(c * 128, 128)]
                g = v > m[c]
                m[c] = jnp.where(g, v, m[c])
                ix[c] = jnp.where(g, j * (_TCCH // 8) + a, ix[c])
            return tuple(m), tuple(ix)

        acc = lax.fori_loop(0, _TCCH // 8, inner, acc)
        if t + _TRING < ntask:
            cps[t + _TRING] = start(t + _TRING)
            cps[t + _TRING].start()
        if j == 3:
            pairs = []
            for c in range(_TUNR):
                pairs.append((acc[0][c],
                              (acc[1][c] * 8 + rows8) * _W
                              + c * 128 + cols128))
            while len(pairs) > 1:
                nxt = []
                for a in range(0, len(pairs), 2):
                    (m1, f1), (m2, f2) = pairs[a], pairs[a + 1]
                    take = (m2 > m1) | ((m2 == m1) & (f2 < f1))
                    nxt.append((jnp.where(take, m2, m1),
                                jnp.where(take, f2, f1)))
                pairs = nxt
            m8, f8 = pairs[0]
            mx = jnp.max(m8)
            cand = jnp.where(m8 == mx, f8, jnp.int32(2**31 - 1))
            idx = jnp.min(cand)
            idx_scalars.append(idx)
            acc = ((neg,) * _TUNR, (zero,) * _TUNR)

    iq = lax.broadcasted_iota(jnp.int32, (_QTC,), 0)
    vec = jnp.zeros((_QTC,), jnp.int32)
    for q, s in enumerate(idx_scalars):
        vec = jnp.where(iq == q, s, vec)
    idx_ref[...] = vec


def _tc_gather_body(idx_smem, pr_smem, sc_rec, off_hbm, scl_hbm,
                    pos_ref, scl_ref, rows0, rows1, rows2, sem):
    copies = []
    idxs = []
    for q in range(_QTC):
        idx = idx_smem[q]
        idxs.append(idx)
        h = idx >> 9
        qg = _QSC + q
        copies.append(pltpu.make_async_copy(
            off_hbm.at[pl.ds((qg * 2) * _H + h, 1), :],
            rows0.at[pl.ds(q, 1), :], sem))
        copies.append(pltpu.make_async_copy(
            off_hbm.at[pl.ds((qg * 2 + 1) * _H + h, 1), :],
            rows1.at[pl.ds(q, 1), :], sem))
        copies.append(pltpu.make_async_copy(
            scl_hbm.at[pl.ds(qg * _H + h, 1), :],
            rows2.at[pl.ds(q, 1), :], sem))
    for cp in copies:
        cp.start()
    iq = lax.broadcasted_iota(jnp.int32, (_QTC, 1), 0)
    idxv = jnp.zeros((_QTC, 1), jnp.int32)
    for q, s in enumerate(idxs):
        idxv = jnp.where(iq == q, s, idxv)
    for cp in copies:
        cp.wait()
    iw = lax.broadcasted_iota(jnp.int32, (_QTC, _W), 1)
    wq = idxv & (_W - 1)
    hq = idxv >> 9
    sel = (iw == wq).astype(jnp.float32)
    o0 = jnp.sum(rows0[...] * sel, axis=1, keepdims=True)
    o1 = jnp.sum(rows1[...] * sel, axis=1, keepdims=True)
    sv = jnp.sum(rows2[...] * sel, axis=1, keepdims=True)
    xs = wq.astype(jnp.float32) + o0
    ys = hq.astype(jnp.float32) + o1
    sc = jnp.exp(sv * jnp.float32(_LN2))
    rec = sc_rec[...]
    pos = jnp.concatenate(
        [jnp.concatenate([rec[:, 0:1], rec[:, 1:2]], axis=1),
         jnp.concatenate([xs, ys], axis=1)], axis=0)
    pr = pr_smem[0]
    pos_ref[...] = (pos + 0.5) * pr - 0.5
    scl_ref[...] = jnp.concatenate([rec[:, 2:3], sc], axis=0)


@jax.jit
def _detect(scores2, off2, scl2, pr):
    mesh = plsc.VectorSubcoreMesh(
        core_axis_name="c", subcore_axis_name="s",
        num_cores=_NC, num_subcores=_NS)
    sc_run = functools.partial(
        pl.kernel,
        out_type=jax.ShapeDtypeStruct((_NW * _L,), jnp.float32),
        mesh=mesh,
        scratch_types=[
            pltpu.VMEM((_RPC, _W), jnp.float32),
            pltpu.VMEM((_RPC, _W), jnp.float32),
            pltpu.VMEM((_W,), jnp.float32),
            pltpu.VMEM((_W,), jnp.float32),
            pltpu.VMEM((_W,), jnp.float32),
            pltpu.VMEM((_L,), jnp.float32),
            pltpu.SemaphoreType.DMA,
            pltpu.SemaphoreType.DMA,
            pltpu.SemaphoreType.DMA,
        ],
        compiler_params=pltpu.CompilerParams(needs_layout_passes=False),
    )(_sc_body)
    out_sc = sc_run(scores2, off2, scl2)

    idx_tc = pl.pallas_call(
        _tc_argmax_body,
        in_specs=[pl.BlockSpec(memory_space=pl.ANY)],
        out_specs=pl.BlockSpec((_QTC,), lambda: (0,)),
        out_shape=jax.ShapeDtypeStruct((_QTC,), jnp.int32),
        scratch_shapes=(
            [pltpu.VMEM((_TCCH, _W), jnp.float32)] * _TRING
            + [pltpu.SemaphoreType.DMA] * _TRING
        ),
    )(scores2)

    positions, sel_scales = pl.pallas_call(
        _tc_gather_body,
        in_specs=[
            pl.BlockSpec(memory_space=pltpu.SMEM),
            pl.BlockSpec(memory_space=pltpu.SMEM),
            pl.BlockSpec((_QSC, _L), lambda: (0, 0)),
            pl.BlockSpec(memory_space=pl.ANY),
            pl.BlockSpec(memory_space=pl.ANY),
        ],
        out_specs=(pl.BlockSpec((_QSC + _QTC, 2), lambda: (0, 0)),
                   pl.BlockSpec((_QSC + _QTC, 1), lambda: (0, 0))),
        out_shape=(jax.ShapeDtypeStruct((_QSC + _QTC, 2), jnp.float32),
                   jax.ShapeDtypeStruct((_QSC + _QTC, 1), jnp.float32)),
        scratch_shapes=[
            pltpu.VMEM((_QTC, _W), jnp.float32),
            pltpu.VMEM((_QTC, _W), jnp.float32),
            pltpu.VMEM((_QTC, _W), jnp.float32),
            pltpu.SemaphoreType.DMA,
        ],
    )(idx_tc, pr, out_sc.reshape(_QSC, _L), off2, scl2)

    return positions, sel_scales.reshape(_QSC + _QTC)


def kernel(scores, scales, offsets, pool_ratio):
    qn = scores.shape[0]
    scores2 = scores.reshape(qn * _H, _W)
    off2 = offsets.reshape(qn * 2 * _H, _W)
    scl2 = scales.reshape(qn * _H, _W)
    pf = jnp.asarray(pool_ratio, jnp.float32).reshape(1)
    return _detect(scores2, off2, scl2, pf)
